# baseline (speedup 1.0000x reference)
import functools
import jax
import jax.numpy as jnp
from jax import lax
from jax.experimental import pallas as pl
from jax.experimental.pallas import tpu as pltpu
from jax.experimental.pallas import tpu_sc as plsc

_NUM_CLASSES = 3
_PC = [-15.0, -30.0, -2.0, 15.0, 30.0, 2.0]
_CLS_W = 2.0
_PTS_W = 5.0
_DIR_W = 0.005
_ALPHA = 0.25
_GAMMA = 2.0
_EPS = 1e-12

_B, _NQ, _G, _S, _P = 4, 512, 32, 20, 20
_D = 2 * _P
_GS = _G * _S
_QT = 32
_HIGH = jax.lax.Precision.HIGHEST

_QW = _NQ // 8
_HI_X = 0.5334
_HI_Y = 0.5167


def _sc_corr_body(ptsf_hbm, gtf_hbm, corr_hbm, ptsf_v, gtf_v, row_v):
    iota16 = lax.iota(jnp.int32, 16)
    wid = lax.axis_index("s") * 2 + lax.axis_index("c")
    b = wid // 8
    qbase = (wid % 8) * _QW

    pltpu.sync_copy(ptsf_hbm.at[b, pl.ds(qbase * _D, _QW * _D)], ptsf_v)
    pltpu.sync_copy(gtf_hbm.at[b], gtf_v)

    zero = jnp.zeros((16,), jnp.float32)

    def zbody(q, _):
        for c in range(_GS // 16):
            row_v[q, pl.ds(c * 16, 16)] = zero
        return 0

    lax.fori_loop(0, _QW, zbody, 0)

    hi_vec = jnp.where(iota16 % 2 == 0, _HI_X, _HI_Y).astype(jnp.float32)

    def scan_body(t, _):
        q = t // 3
        tm = t % 3
        c16 = tm * 16 - 8 * (tm // 2)
        chunk = ptsf_v[pl.ds(q * _D + c16, 16)]
        lowlane = jnp.where(tm == 2, 8, 0)
        band = (chunk >= 0.5) & (chunk < hi_vec) & (iota16 >= lowlane)
        cnt = plsc.all_reduce_population_count(band)[0]

        def hits(_i, mvec):
            maskb = mvec > 0
            lsel_v = plsc.all_reduce_ffs(maskb)
            lanesel = lsel_v[0]
            d = c16 + lanesel
            par = lanesel % 2
            offd = jnp.where(par == 0, _PC[0], _PC[1]).astype(jnp.float32)
            scld = jnp.where(par == 0, _PC[3] - _PC[0],
                             _PC[4] - _PC[1]).astype(jnp.float32)
            fac = jnp.where(par == 0, 2.0 / (_PC[3] - _PC[0]),
                            2.0 / (_PC[4] - _PC[1])).astype(jnp.float32)
            p_s = chunk.at[lsel_v].get(mode="promise_in_bounds")[0]
            tq = offd + scld * p_s
            for c in range(_GS // 16):
                gch = gtf_v[pl.ds(d * _GS + c * 16, 16)]
                r = row_v[q, pl.ds(c * 16, 16)]
                row_v[q, pl.ds(c * 16, 16)] = (
                    r + fac * jnp.maximum(gch - tq, 0.0))
            return jnp.where(iota16 == lanesel, 0, mvec)

        @pl.when(cnt > 0)
        def _():
            lax.fori_loop(0, cnt, hits,
                          jnp.where(band, 1, 0).astype(jnp.int32))

        return 0

    lax.fori_loop(0, _QW * 3, scan_body, 0)
    pltpu.sync_copy(row_v, corr_hbm.at[b, pl.ds(qbase, _QW)])


def _sc_corr(pts, gt_t):
    mesh = plsc.VectorSubcoreMesh(core_axis_name="c", subcore_axis_name="s")
    kern = pl.kernel(
        _sc_corr_body,
        mesh=mesh,
        compiler_params=pltpu.CompilerParams(needs_layout_passes=False),
        out_type=jax.ShapeDtypeStruct((_B, _NQ, _GS), jnp.float32),
        scratch_types=[
            pltpu.VMEM((_QW * _D,), jnp.float32),
            pltpu.VMEM((_D * _GS,), jnp.float32),
            pltpu.VMEM((_QW, _GS), jnp.float32),
        ],
    )
    return kern(pts.reshape(_B, _NQ * _D), gt_t.reshape(_B, _D * _GS))


def _body(cls_ref, pts_ref, gtt_ref, lab_ref, corr_ref, out_ref,
          gtn_ref, m_ref, ord_ref, sext_ref, gext_ref):
    pw = _PC[3] - _PC[0]
    ph = _PC[4] - _PC[1]

    drow = jax.lax.broadcasted_iota(jnp.int32, (_D, 1), 0)
    off_d = jnp.where(drow % 2 == 0, _PC[0], _PC[1]).astype(jnp.float32)
    scl_d = jnp.where(drow % 2 == 0, pw, ph).astype(jnp.float32)
    gtn = (gtt_ref[0] - off_d) / scl_d
    gtn_ref[...] = gtn

    pts_full = pts_ref[0]
    sgn = jnp.where(pts_full >= 0.5, 1.0, -1.0)
    rowterm = jnp.sum(sgn * pts_full, axis=1, keepdims=True)
    sext_ref[...] = jnp.concatenate(
        [sgn, rowterm, jnp.zeros((_NQ, 7), jnp.float32)], axis=1)
    grow = jax.lax.broadcasted_iota(jnp.int32, (8, _GS), 0)
    gext_ref[0:_D, :] = gtn
    gext_ref[_D:_D + 8, :] = jnp.where(grow == 0, -1.0, 0.0)

    s_low = (jax.lax.broadcasted_iota(jnp.int32, (_QT, 128), 1) // _G)
    s_low = s_low.astype(jnp.float32)

    def one_tile(base):
        st = sext_ref[pl.ds(base, _QT), :]
        mm = jax.lax.dot_general(st, gext_ref[...], (((1,), (0,)), ((), ())),
                                 precision=_HIGH,
                                 preferred_element_type=jnp.float32)
        acc = corr_ref[0, pl.ds(base, _QT), :] - mm
        m = acc[:, 0:128]
        sv = s_low
        for k in range(1, 5):
            sl = acc[:, 128 * k:128 * (k + 1)]
            upd = sl < m
            m = jnp.where(upd, sl, m)
            sv = jnp.where(upd, s_low + jnp.float32(4 * k), sv)
        for sh in (64, 96):
            mr = pltpu.roll(m, sh, 1)
            sr = pltpu.roll(sv, sh, 1)
            take = (mr < m) | ((mr == m) & (sr < sv))
            m = jnp.where(take, mr, m)
            sv = jnp.where(take, sr, sv)
        m_ref[pl.ds(base, _QT), :] = m[:, 0:_G]
        ord_ref[pl.ds(base, _QT), :] = sv[:, 0:_G]

    def qt_body(i, _):
        one_tile(i * 2 * _QT)
        one_tile(i * 2 * _QT + _QT)
        return 0

    jax.lax.fori_loop(0, _NQ // (2 * _QT), qt_body, 0)

    x = cls_ref[0]
    p = jax.nn.sigmoid(x)
    negc = -jnp.log(1.0 - p + _EPS) * (p ** _GAMMA) * (1.0 - _ALPHA)
    posc = -jnp.log(p + _EPS) * ((1.0 - p) ** _GAMMA) * _ALPHA
    clsfull = posc - negc
    neg_sum = jnp.sum(negc)

    lab = lab_ref[0]
    cidx = jax.lax.broadcasted_iota(jnp.int32, (_NUM_CLASSES, _G), 0)
    oh3 = jnp.where(cidx == lab, 1.0, 0.0)
    clscol = jax.lax.dot_general(clsfull, oh3, (((1,), (0,)), ((), ())),
                                 precision=_HIGH,
                                 preferred_element_type=jnp.float32)

    m_full = m_ref[...]
    cost = _CLS_W * clscol + (_PTS_W / _P) * m_full

    minv = jnp.min(cost, axis=0, keepdims=True)
    qid = jax.lax.broadcasted_iota(jnp.int32, (_NQ, _G), 0).astype(jnp.float32)
    aq = jnp.min(jnp.where(cost == minv, qid, jnp.float32(_NQ)),
                 axis=0, keepdims=True)

    id32 = jnp.where(
        jax.lax.broadcasted_iota(jnp.int32, (_G, _G), 0)
        == jax.lax.broadcasted_iota(jnp.int32, (_G, _G), 1), 1.0, 0.0)
    aq_col = jax.lax.dot_general(id32, aq, (((1,), (1,)), ((), ())),
                                 precision=_HIGH,
                                 preferred_element_type=jnp.float32)
    later = (jax.lax.broadcasted_iota(jnp.int32, (_G, _G), 0)
             > jax.lax.broadcasted_iota(jnp.int32, (_G, _G), 1))
    eqm = jnp.where((aq_col == aq) & later, 1.0, 0.0)
    dup = jnp.max(eqm, axis=0, keepdims=True)
    win = 1.0 - dup

    oh = jnp.where(qid == aq, 1.0, 0.0)
    m_at = jnp.sum(oh * m_full, axis=0, keepdims=True)
    shift = jnp.sum(oh * ord_ref[...], axis=0, keepdims=True)
    cls_at = (minv - (_PTS_W / _P) * m_at) * (1.0 / _CLS_W)

    pos_cnt = jnp.sum(win)
    cls_num = _CLS_W * (neg_sum + jnp.sum(win * cls_at))
    pts_num = _PTS_W * jnp.sum(win * m_at)

    predpts = jax.lax.dot_general(oh, pts_full, (((0,), (0,)), ((), ())),
                                  preferred_element_type=jnp.float32)
    gsrow = jax.lax.broadcasted_iota(jnp.int32, (_GS, 1), 0)
    gmod = (gsrow & (_G - 1)).astype(jnp.float32)
    gidx = jax.lax.broadcasted_iota(jnp.int32, (1, _G), 1).astype(jnp.float32)
    sdiv = jax.lax.shift_right_logical(gsrow, 5).astype(jnp.float32)
    ohs = jnp.where((gmod == gidx) & (sdiv == shift), 1.0, 0.0)
    tgt = jax.lax.dot_general(ohs, gtn_ref[...], (((0,), (1,)), ((), ())),
                              preferred_element_type=jnp.float32)

    dcol = jax.lax.broadcasted_iota(jnp.int32, (1, _D), 1)
    off_l = jnp.where(dcol % 2 == 0, _PC[0], _PC[1]).astype(jnp.float32)
    scl_l = jnp.where(dcol % 2 == 0, pw, ph).astype(jnp.float32)
    pred_den = predpts * scl_l + off_l
    tgt_den = tgt * scl_l + off_l

    pd = pred_den[:, 2:_D] - pred_den[:, 0:_D - 2]
    td = tgt_den[:, 2:_D] - tgt_den[:, 0:_D - 2]
    prod = pd * td
    pp = pd * pd
    tt = td * td
    dot2 = prod[:, 0:_D - 3] + prod[:, 1:_D - 2]
    pp2 = pp[:, 0:_D - 3] + pp[:, 1:_D - 2]
    tt2 = tt[:, 0:_D - 3] + tt[:, 1:_D - 2]
    cos = dot2 / (jnp.sqrt(pp2) * jnp.sqrt(tt2) + _EPS)
    seg_lane = jax.lax.broadcasted_iota(jnp.int32, (_G, _D - 3), 1)
    valid = (seg_lane % 2) == 0
    win_col = jax.lax.dot_general(id32, win, (((1,), (1,)), ((), ())),
                                  preferred_element_type=jnp.float32)
    dir_num = _DIR_W * jnp.sum(jnp.where(valid, (1.0 - cos) * win_col, 0.0))

    lane = jax.lax.broadcasted_iota(jnp.int32, (1, 128), 1)
    row = (jnp.where(lane == 0, pos_cnt, 0.0)
           + jnp.where(lane == 1, cls_num, 0.0)
           + jnp.where(lane == 2, pts_num, 0.0)
           + jnp.where(lane == 3, dir_num, 0.0))
    out_ref[0] = row


@jax.jit
def kernel(cls_scores, pts_preds, gt_shifts_pts, gt_labels):
    pts = pts_preds.reshape(_B, _NQ, _D)
    gt_t = jnp.transpose(gt_shifts_pts.reshape(_B, _G, _S, _D),
                         (0, 3, 2, 1)).reshape(_B, _D, _GS)
    lab = gt_labels.astype(jnp.int32).reshape(_B, 1, _G)

    corr = _sc_corr(pts, gt_t)

    out = pl.pallas_call(
        _body,
        grid=(_B,),
        in_specs=[
            pl.BlockSpec((1, _NQ, _NUM_CLASSES), lambda b: (b, 0, 0)),
            pl.BlockSpec((1, _NQ, _D), lambda b: (b, 0, 0)),
            pl.BlockSpec((1, _D, _GS), lambda b: (b, 0, 0)),
            pl.BlockSpec((1, 1, _G), lambda b: (b, 0, 0)),
            pl.BlockSpec((1, _NQ, _GS), lambda b: (b, 0, 0)),
        ],
        out_specs=pl.BlockSpec((1, 1, 128), lambda b: (b, 0, 0)),
        out_shape=jax.ShapeDtypeStruct((_B, 1, 128), jnp.float32),
        scratch_shapes=[
            pltpu.VMEM((_D, _GS), jnp.float32),
            pltpu.VMEM((_NQ, _G), jnp.float32),
            pltpu.VMEM((_NQ, _G), jnp.float32),
            pltpu.VMEM((_NQ, 48), jnp.float32),
            pltpu.VMEM((48, _GS), jnp.float32),
        ],
    )(cls_scores, pts, gt_t, lab, corr)

    s = jnp.sum(out[:, 0, :4], axis=0)
    num_pos = jnp.maximum(s[0], 1.0)
    return (s[1] + s[2] + s[3]) / num_pos


# speedup vs baseline: 1.0183x; 1.0183x over previous
import functools
import jax
import jax.numpy as jnp
from jax import lax
from jax.experimental import pallas as pl
from jax.experimental.pallas import tpu as pltpu
from jax.experimental.pallas import tpu_sc as plsc

_NUM_CLASSES = 3
_PC = [-15.0, -30.0, -2.0, 15.0, 30.0, 2.0]
_CLS_W = 2.0
_PTS_W = 5.0
_DIR_W = 0.005
_ALPHA = 0.25
_GAMMA = 2.0
_EPS = 1e-12

_B, _NQ, _G, _S, _P = 4, 512, 32, 20, 20
_D = 2 * _P
_GS = _G * _S
_QT = 32
_HIGH = jax.lax.Precision.HIGHEST

_QW = _NQ // 8
_HI_X = 0.5334
_HI_Y = 0.5167


def _sc_corr_body(ptsf_hbm, gtf_hbm, corr_hbm, ptsf_v, gtf_v, row_v,
                  sem_pts, sem_gt):
    iota16 = lax.iota(jnp.int32, 16)
    wid = lax.axis_index("s") * 2 + lax.axis_index("c")
    b = wid // 8
    qbase = (wid % 8) * _QW

    cp_pts = pltpu.async_copy(
        ptsf_hbm.at[b, pl.ds(qbase * _D, _QW * _D)], ptsf_v, sem_pts)
    cp_gt = pltpu.async_copy(gtf_hbm.at[b], gtf_v, sem_gt)

    zero = jnp.zeros((16,), jnp.float32)

    def zbody(q, _):
        for c in range(_GS // 16):
            row_v[q, pl.ds(c * 16, 16)] = zero
        return 0

    lax.fori_loop(0, _QW, zbody, 0)
    cp_pts.wait()
    cp_gt.wait()

    hi_vec = jnp.where(iota16 % 2 == 0, _HI_X, _HI_Y).astype(jnp.float32)

    def scan_body(t, _):
        q = t // 3
        tm = t % 3
        c16 = tm * 16 - 8 * (tm // 2)
        chunk = ptsf_v[pl.ds(q * _D + c16, 16)]
        lowlane = jnp.where(tm == 2, 8, 0)
        band = (chunk >= 0.5) & (chunk < hi_vec) & (iota16 >= lowlane)
        cnt = plsc.all_reduce_population_count(band)[0]

        def hits(_i, mvec):
            maskb = mvec > 0
            lsel_v = plsc.all_reduce_ffs(maskb)
            lanesel = lsel_v[0]
            d = c16 + lanesel
            par = lanesel % 2
            offd = jnp.where(par == 0, _PC[0], _PC[1]).astype(jnp.float32)
            scld = jnp.where(par == 0, _PC[3] - _PC[0],
                             _PC[4] - _PC[1]).astype(jnp.float32)
            fac = jnp.where(par == 0, 2.0 / (_PC[3] - _PC[0]),
                            2.0 / (_PC[4] - _PC[1])).astype(jnp.float32)
            p_s = chunk.at[lsel_v].get(mode="promise_in_bounds")[0]
            tq = offd + scld * p_s
            for c in range(_GS // 16):
                gch = gtf_v[pl.ds(d * _GS + c * 16, 16)]
                r = row_v[q, pl.ds(c * 16, 16)]
                row_v[q, pl.ds(c * 16, 16)] = (
                    r + fac * jnp.maximum(gch - tq, 0.0))
            return jnp.where(iota16 == lanesel, 0, mvec)

        @pl.when(cnt > 0)
        def _():
            lax.fori_loop(0, cnt, hits,
                          jnp.where(band, 1, 0).astype(jnp.int32))

        return 0

    lax.fori_loop(0, _QW * 3, scan_body, 0)
    pltpu.sync_copy(row_v, corr_hbm.at[b, pl.ds(qbase, _QW)])


def _sc_corr(pts, gt_t):
    mesh = plsc.VectorSubcoreMesh(core_axis_name="c", subcore_axis_name="s")
    kern = pl.kernel(
        _sc_corr_body,
        mesh=mesh,
        compiler_params=pltpu.CompilerParams(needs_layout_passes=False),
        out_type=jax.ShapeDtypeStruct((_B, _NQ, _GS), jnp.float32),
        scratch_types=[
            pltpu.VMEM((_QW * _D,), jnp.float32),
            pltpu.VMEM((_D * _GS,), jnp.float32),
            pltpu.VMEM((_QW, _GS), jnp.float32),
            pltpu.SemaphoreType.DMA,
            pltpu.SemaphoreType.DMA,
        ],
    )
    return kern(pts.reshape(_B, _NQ * _D), gt_t.reshape(_B, _D * _GS))


def _body(cls_ref, pts_ref, gtt_ref, lab_ref, corr_ref, out_ref,
          gtn_ref, m_ref, ord_ref, sext_ref, gext_ref):
    pw = _PC[3] - _PC[0]
    ph = _PC[4] - _PC[1]

    drow = jax.lax.broadcasted_iota(jnp.int32, (_D, 1), 0)
    off_d = jnp.where(drow % 2 == 0, _PC[0], _PC[1]).astype(jnp.float32)
    scl_d = jnp.where(drow % 2 == 0, pw, ph).astype(jnp.float32)
    gtn = (gtt_ref[0] - off_d) / scl_d
    gtn_ref[...] = gtn

    pts_full = pts_ref[0]
    sgn = jnp.where(pts_full >= 0.5, 1.0, -1.0)
    rowterm = jnp.sum(sgn * pts_full, axis=1, keepdims=True)
    sext_ref[...] = jnp.concatenate(
        [sgn, rowterm, jnp.zeros((_NQ, 7), jnp.float32)], axis=1)
    grow = jax.lax.broadcasted_iota(jnp.int32, (8, _GS), 0)
    gext_ref[0:_D, :] = gtn
    gext_ref[_D:_D + 8, :] = jnp.where(grow == 0, -1.0, 0.0)

    s_low = (jax.lax.broadcasted_iota(jnp.int32, (_QT, 128), 1) // _G)
    s_low = s_low.astype(jnp.float32)

    def one_tile(base):
        st = sext_ref[pl.ds(base, _QT), :]
        mm = jax.lax.dot_general(st, gext_ref[...], (((1,), (0,)), ((), ())),
                                 precision=_HIGH,
                                 preferred_element_type=jnp.float32)
        acc = corr_ref[0, pl.ds(base, _QT), :] - mm
        m = acc[:, 0:128]
        sv = s_low
        for k in range(1, 5):
            sl = acc[:, 128 * k:128 * (k + 1)]
            upd = sl < m
            m = jnp.where(upd, sl, m)
            sv = jnp.where(upd, s_low + jnp.float32(4 * k), sv)
        for sh in (64, 96):
            mr = pltpu.roll(m, sh, 1)
            sr = pltpu.roll(sv, sh, 1)
            take = (mr < m) | ((mr == m) & (sr < sv))
            m = jnp.where(take, mr, m)
            sv = jnp.where(take, sr, sv)
        m_ref[pl.ds(base, _QT), :] = m[:, 0:_G]
        ord_ref[pl.ds(base, _QT), :] = sv[:, 0:_G]

    def qt_body(i, _):
        one_tile(i * 2 * _QT)
        one_tile(i * 2 * _QT + _QT)
        return 0

    jax.lax.fori_loop(0, _NQ // (2 * _QT), qt_body, 0)

    x = cls_ref[0]
    p = jax.nn.sigmoid(x)
    negc = -jnp.log(1.0 - p + _EPS) * (p ** _GAMMA) * (1.0 - _ALPHA)
    posc = -jnp.log(p + _EPS) * ((1.0 - p) ** _GAMMA) * _ALPHA
    clsfull = posc - negc
    neg_sum = jnp.sum(negc)

    lab = lab_ref[0]
    cidx = jax.lax.broadcasted_iota(jnp.int32, (_NUM_CLASSES, _G), 0)
    oh3 = jnp.where(cidx == lab, 1.0, 0.0)
    clscol = jax.lax.dot_general(clsfull, oh3, (((1,), (0,)), ((), ())),
                                 precision=_HIGH,
                                 preferred_element_type=jnp.float32)

    m_full = m_ref[...]
    cost = _CLS_W * clscol + (_PTS_W / _P) * m_full

    minv = jnp.min(cost, axis=0, keepdims=True)
    qid = jax.lax.broadcasted_iota(jnp.int32, (_NQ, _G), 0).astype(jnp.float32)
    aq = jnp.min(jnp.where(cost == minv, qid, jnp.float32(_NQ)),
                 axis=0, keepdims=True)

    id32 = jnp.where(
        jax.lax.broadcasted_iota(jnp.int32, (_G, _G), 0)
        == jax.lax.broadcasted_iota(jnp.int32, (_G, _G), 1), 1.0, 0.0)
    aq_col = jax.lax.dot_general(id32, aq, (((1,), (1,)), ((), ())),
                                 precision=_HIGH,
                                 preferred_element_type=jnp.float32)
    later = (jax.lax.broadcasted_iota(jnp.int32, (_G, _G), 0)
             > jax.lax.broadcasted_iota(jnp.int32, (_G, _G), 1))
    eqm = jnp.where((aq_col == aq) & later, 1.0, 0.0)
    dup = jnp.max(eqm, axis=0, keepdims=True)
    win = 1.0 - dup

    oh = jnp.where(qid == aq, 1.0, 0.0)
    m_at = jnp.sum(oh * m_full, axis=0, keepdims=True)
    shift = jnp.sum(oh * ord_ref[...], axis=0, keepdims=True)
    cls_at = (minv - (_PTS_W / _P) * m_at) * (1.0 / _CLS_W)

    pos_cnt = jnp.sum(win)
    cls_num = _CLS_W * (neg_sum + jnp.sum(win * cls_at))
    pts_num = _PTS_W * jnp.sum(win * m_at)

    predpts = jax.lax.dot_general(oh, pts_full, (((0,), (0,)), ((), ())),
                                  preferred_element_type=jnp.float32)
    gsrow = jax.lax.broadcasted_iota(jnp.int32, (_GS, 1), 0)
    gmod = (gsrow & (_G - 1)).astype(jnp.float32)
    gidx = jax.lax.broadcasted_iota(jnp.int32, (1, _G), 1).astype(jnp.float32)
    sdiv = jax.lax.shift_right_logical(gsrow, 5).astype(jnp.float32)
    ohs = jnp.where((gmod == gidx) & (sdiv == shift), 1.0, 0.0)
    tgt = jax.lax.dot_general(ohs, gtn_ref[...], (((0,), (1,)), ((), ())),
                              preferred_element_type=jnp.float32)

    dcol = jax.lax.broadcasted_iota(jnp.int32, (1, _D), 1)
    off_l = jnp.where(dcol % 2 == 0, _PC[0], _PC[1]).astype(jnp.float32)
    scl_l = jnp.where(dcol % 2 == 0, pw, ph).astype(jnp.float32)
    pred_den = predpts * scl_l + off_l
    tgt_den = tgt * scl_l + off_l

    pd = pred_den[:, 2:_D] - pred_den[:, 0:_D - 2]
    td = tgt_den[:, 2:_D] - tgt_den[:, 0:_D - 2]
    prod = pd * td
    pp = pd * pd
    tt = td * td
    dot2 = prod[:, 0:_D - 3] + prod[:, 1:_D - 2]
    pp2 = pp[:, 0:_D - 3] + pp[:, 1:_D - 2]
    tt2 = tt[:, 0:_D - 3] + tt[:, 1:_D - 2]
    cos = dot2 / (jnp.sqrt(pp2) * jnp.sqrt(tt2) + _EPS)
    seg_lane = jax.lax.broadcasted_iota(jnp.int32, (_G, _D - 3), 1)
    valid = (seg_lane % 2) == 0
    win_col = jax.lax.dot_general(id32, win, (((1,), (1,)), ((), ())),
                                  preferred_element_type=jnp.float32)
    dir_num = _DIR_W * jnp.sum(jnp.where(valid, (1.0 - cos) * win_col, 0.0))

    lane = jax.lax.broadcasted_iota(jnp.int32, (1, 128), 1)
    row = (jnp.where(lane == 0, pos_cnt, 0.0)
           + jnp.where(lane == 1, cls_num, 0.0)
           + jnp.where(lane == 2, pts_num, 0.0)
           + jnp.where(lane == 3, dir_num, 0.0))
    out_ref[0] = row


@jax.jit
def kernel(cls_scores, pts_preds, gt_shifts_pts, gt_labels):
    pts = pts_preds.reshape(_B, _NQ, _D)
    gt_t = jnp.transpose(gt_shifts_pts.reshape(_B, _G, _S, _D),
                         (0, 3, 2, 1)).reshape(_B, _D, _GS)
    lab = gt_labels.astype(jnp.int32).reshape(_B, 1, _G)

    corr = _sc_corr(pts, gt_t)

    out = pl.pallas_call(
        _body,
        grid=(_B,),
        in_specs=[
            pl.BlockSpec((1, _NQ, _NUM_CLASSES), lambda b: (b, 0, 0)),
            pl.BlockSpec((1, _NQ, _D), lambda b: (b, 0, 0)),
            pl.BlockSpec((1, _D, _GS), lambda b: (b, 0, 0)),
            pl.BlockSpec((1, 1, _G), lambda b: (b, 0, 0)),
            pl.BlockSpec((1, _NQ, _GS), lambda b: (b, 0, 0)),
        ],
        out_specs=pl.BlockSpec((1, 1, 128), lambda b: (b, 0, 0)),
        out_shape=jax.ShapeDtypeStruct((_B, 1, 128), jnp.float32),
        scratch_shapes=[
            pltpu.VMEM((_D, _GS), jnp.float32),
            pltpu.VMEM((_NQ, _G), jnp.float32),
            pltpu.VMEM((_NQ, _G), jnp.float32),
            pltpu.VMEM((_NQ, 48), jnp.float32),
            pltpu.VMEM((48, _GS), jnp.float32),
        ],
    )(cls_scores, pts, gt_t, lab, corr)

    s = jnp.sum(out[:, 0, :4], axis=0)
    num_pos = jnp.maximum(s[0], 1.0)
    return (s[1] + s[2] + s[3]) / num_pos


# speedup vs baseline: 2.0155x; 1.9792x over previous
import functools
import jax
import jax.numpy as jnp
from jax.experimental import pallas as pl
from jax.experimental.pallas import tpu as pltpu

_NUM_CLASSES = 3
_PC = [-15.0, -30.0, -2.0, 15.0, 30.0, 2.0]
_CLS_W = 2.0
_PTS_W = 5.0
_DIR_W = 0.005
_ALPHA = 0.25
_GAMMA = 2.0
_EPS = 1e-12

_B, _NQ, _G, _S, _P = 4, 512, 32, 20, 20
_D = 2 * _P
_GS = _G * _S
_QT = 32
_HIGH = jax.lax.Precision.HIGHEST


def _body(cls_ref, pts_ref, gtt_ref, lab_ref, out_ref, gtn_ref, m_ref, ord_ref):
    pw = _PC[3] - _PC[0]
    ph = _PC[4] - _PC[1]

    drow = jax.lax.broadcasted_iota(jnp.int32, (_D, 1), 0)
    off_d = jnp.where(drow % 2 == 0, _PC[0], _PC[1]).astype(jnp.float32)
    scl_d = jnp.where(drow % 2 == 0, pw, ph).astype(jnp.float32)
    gtn_ref[...] = (gtt_ref[0] - off_d) / scl_d

    s_low = (jax.lax.broadcasted_iota(jnp.int32, (_QT, 128), 1) // _G)
    s_low = s_low.astype(jnp.float32)

    def one_tile(base):
        pts_t = pts_ref[0, pl.ds(base, _QT), :]
        acc = jnp.zeros((_QT, _GS), jnp.float32)
        for d in range(_D):
            acc = acc + jnp.abs(pts_t[:, d:d + 1] - gtn_ref[d:d + 1, :])
        m = acc[:, 0:128]
        sv = s_low
        for k in range(1, 5):
            sl = acc[:, 128 * k:128 * (k + 1)]
            upd = sl < m
            m = jnp.where(upd, sl, m)
            sv = jnp.where(upd, s_low + jnp.float32(4 * k), sv)
        for sh in (64, 96):
            mr = pltpu.roll(m, sh, 1)
            sr = pltpu.roll(sv, sh, 1)
            take = (mr < m) | ((mr == m) & (sr < sv))
            m = jnp.where(take, mr, m)
            sv = jnp.where(take, sr, sv)
        m_ref[pl.ds(base, _QT), :] = m[:, 0:_G]
        ord_ref[pl.ds(base, _QT), :] = sv[:, 0:_G]

    for i in range(_NQ // _QT):
        one_tile(i * _QT)

    x = cls_ref[0]
    p = jax.nn.sigmoid(x)
    negc = -jnp.log(1.0 - p + _EPS) * (p ** _GAMMA) * (1.0 - _ALPHA)
    posc = -jnp.log(p + _EPS) * ((1.0 - p) ** _GAMMA) * _ALPHA
    clsfull = posc - negc
    neg_sum = jnp.sum(negc)

    lab = lab_ref[0]
    cidx = jax.lax.broadcasted_iota(jnp.int32, (_NUM_CLASSES, _G), 0)
    oh3 = jnp.where(cidx == lab, 1.0, 0.0)
    clscol = jax.lax.dot_general(clsfull, oh3, (((1,), (0,)), ((), ())),
                                 precision=_HIGH,
                                 preferred_element_type=jnp.float32)

    m_full = m_ref[...]
    cost = _CLS_W * clscol + (_PTS_W / _P) * m_full

    minv = jnp.min(cost, axis=0, keepdims=True)
    qid = jax.lax.broadcasted_iota(jnp.int32, (_NQ, _G), 0).astype(jnp.float32)
    aq = jnp.min(jnp.where(cost == minv, qid, jnp.float32(_NQ)),
                 axis=0, keepdims=True)

    id32 = jnp.where(
        jax.lax.broadcasted_iota(jnp.int32, (_G, _G), 0)
        == jax.lax.broadcasted_iota(jnp.int32, (_G, _G), 1), 1.0, 0.0)
    aq_col = jax.lax.dot_general(id32, aq, (((1,), (1,)), ((), ())),
                                 precision=_HIGH,
                                 preferred_element_type=jnp.float32)
    later = (jax.lax.broadcasted_iota(jnp.int32, (_G, _G), 0)
             > jax.lax.broadcasted_iota(jnp.int32, (_G, _G), 1))
    eqm = jnp.where((aq_col == aq) & later, 1.0, 0.0)
    dup = jnp.max(eqm, axis=0, keepdims=True)
    win = 1.0 - dup

    oh = jnp.where(qid == aq, 1.0, 0.0)
    m_at = jnp.sum(oh * m_full, axis=0, keepdims=True)
    shift = jnp.sum(oh * ord_ref[...], axis=0, keepdims=True)
    cls_at = (minv - (_PTS_W / _P) * m_at) * (1.0 / _CLS_W)

    pos_cnt = jnp.sum(win)
    cls_num = _CLS_W * (neg_sum + jnp.sum(win * cls_at))
    pts_num = _PTS_W * jnp.sum(win * m_at)

    predpts = jax.lax.dot_general(oh, pts_ref[0], (((0,), (0,)), ((), ())),
                                  preferred_element_type=jnp.float32)
    gsrow = jax.lax.broadcasted_iota(jnp.int32, (_GS, 1), 0)
    gmod = (gsrow & (_G - 1)).astype(jnp.float32)
    gidx = jax.lax.broadcasted_iota(jnp.int32, (1, _G), 1).astype(jnp.float32)
    sdiv = jax.lax.shift_right_logical(gsrow, 5).astype(jnp.float32)
    ohs = jnp.where((gmod == gidx) & (sdiv == shift), 1.0, 0.0)
    tgt = jax.lax.dot_general(ohs, gtn_ref[...], (((0,), (1,)), ((), ())),
                              preferred_element_type=jnp.float32)

    dcol = jax.lax.broadcasted_iota(jnp.int32, (1, _D), 1)
    off_l = jnp.where(dcol % 2 == 0, _PC[0], _PC[1]).astype(jnp.float32)
    scl_l = jnp.where(dcol % 2 == 0, pw, ph).astype(jnp.float32)
    pred_den = predpts * scl_l + off_l
    tgt_den = tgt * scl_l + off_l

    pd = pred_den[:, 2:_D] - pred_den[:, 0:_D - 2]
    td = tgt_den[:, 2:_D] - tgt_den[:, 0:_D - 2]
    prod = pd * td
    pp = pd * pd
    tt = td * td
    dot2 = prod[:, 0:_D - 3] + prod[:, 1:_D - 2]
    pp2 = pp[:, 0:_D - 3] + pp[:, 1:_D - 2]
    tt2 = tt[:, 0:_D - 3] + tt[:, 1:_D - 2]
    cos = dot2 / (jnp.sqrt(pp2) * jnp.sqrt(tt2) + _EPS)
    seg_lane = jax.lax.broadcasted_iota(jnp.int32, (_G, _D - 3), 1)
    valid = (seg_lane % 2) == 0
    win_col = jax.lax.dot_general(id32, win, (((1,), (1,)), ((), ())),
                                  preferred_element_type=jnp.float32)
    dir_num = _DIR_W * jnp.sum(
        jnp.where(valid, (1.0 - cos) * win_col, 0.0))

    lane = jax.lax.broadcasted_iota(jnp.int32, (1, 128), 1)
    row = (jnp.where(lane == 0, pos_cnt, 0.0)
           + jnp.where(lane == 1, cls_num, 0.0)
           + jnp.where(lane == 2, pts_num, 0.0)
           + jnp.where(lane == 3, dir_num, 0.0))
    out_ref[0] = row


@jax.jit
def kernel(cls_scores, pts_preds, gt_shifts_pts, gt_labels):
    pts = pts_preds.reshape(_B, _NQ, _D)
    gt_t = jnp.transpose(gt_shifts_pts.reshape(_B, _G, _S, _D),
                         (0, 3, 2, 1)).reshape(_B, _D, _GS)
    lab = gt_labels.astype(jnp.int32).reshape(_B, 1, _G)

    out = pl.pallas_call(
        _body,
        grid=(_B,),
        in_specs=[
            pl.BlockSpec((1, _NQ, _NUM_CLASSES), lambda b: (b, 0, 0)),
            pl.BlockSpec((1, _NQ, _D), lambda b: (b, 0, 0)),
            pl.BlockSpec((1, _D, _GS), lambda b: (b, 0, 0)),
            pl.BlockSpec((1, 1, _G), lambda b: (b, 0, 0)),
        ],
        out_specs=pl.BlockSpec((1, 1, 128), lambda b: (b, 0, 0)),
        out_shape=jax.ShapeDtypeStruct((_B, 1, 128), jnp.float32),
        scratch_shapes=[
            pltpu.VMEM((_D, _GS), jnp.float32),
            pltpu.VMEM((_NQ, _G), jnp.float32),
            pltpu.VMEM((_NQ, _G), jnp.float32),
        ],
    )(cls_scores, pts, gt_t, lab)

    s = jnp.sum(out[:, 0, :4], axis=0)
    num_pos = jnp.maximum(s[0], 1.0)
    return (s[1] + s[2] + s[3]) / num_pos


# speedup vs baseline: 2.2013x; 1.0922x over previous
import functools
import jax
import jax.numpy as jnp
from jax.experimental import pallas as pl
from jax.experimental.pallas import tpu as pltpu

_NUM_CLASSES = 3
_PC = [-15.0, -30.0, -2.0, 15.0, 30.0, 2.0]
_CLS_W = 2.0
_PTS_W = 5.0
_DIR_W = 0.005
_ALPHA = 0.25
_GAMMA = 2.0
_EPS = 1e-12

_B, _NQ, _G, _S, _P = 4, 512, 32, 20, 20
_D = 2 * _P
_GS = _G * _S
_QT = 64
_HIGH = jax.lax.Precision.HIGHEST


def _body(cls_ref, pts_ref, gtt_ref, lab_ref, out_ref, gtn_ref, m_ref, ord_ref):
    pw = _PC[3] - _PC[0]
    ph = _PC[4] - _PC[1]

    drow = jax.lax.broadcasted_iota(jnp.int32, (_D, 1), 0)
    off_d = jnp.where(drow % 2 == 0, _PC[0], _PC[1]).astype(jnp.float32)
    scl_d = jnp.where(drow % 2 == 0, pw, ph).astype(jnp.float32)
    gtn_ref[...] = (gtt_ref[0] - off_d) / scl_d

    s_low = (jax.lax.broadcasted_iota(jnp.int32, (_QT, 128), 1) // _G)
    s_low = s_low.astype(jnp.float32)

    def one_tile(base):
        pts_t = pts_ref[0, pl.ds(base, _QT), :]
        acc = jnp.zeros((_QT, _GS), jnp.float32)
        for d in range(_D):
            acc = acc + jnp.abs(pts_t[:, d:d + 1] - gtn_ref[d:d + 1, :])
        m = acc[:, 0:128]
        sv = s_low
        for k in range(1, 5):
            sl = acc[:, 128 * k:128 * (k + 1)]
            upd = sl < m
            m = jnp.where(upd, sl, m)
            sv = jnp.where(upd, s_low + jnp.float32(4 * k), sv)
        for sh in (64, 96):
            mr = pltpu.roll(m, sh, 1)
            sr = pltpu.roll(sv, sh, 1)
            take = (mr < m) | ((mr == m) & (sr < sv))
            m = jnp.where(take, mr, m)
            sv = jnp.where(take, sr, sv)
        m_ref[pl.ds(base, _QT), :] = m[:, 0:_G]
        ord_ref[pl.ds(base, _QT), :] = sv[:, 0:_G]

    for i in range(_NQ // _QT):
        one_tile(i * _QT)

    x = cls_ref[0]
    p = jax.nn.sigmoid(x)
    negc = -jnp.log(1.0 - p + _EPS) * (p ** _GAMMA) * (1.0 - _ALPHA)
    posc = -jnp.log(p + _EPS) * ((1.0 - p) ** _GAMMA) * _ALPHA
    clsfull = posc - negc
    neg_sum = jnp.sum(negc)

    lab = lab_ref[0]
    cidx = jax.lax.broadcasted_iota(jnp.int32, (_NUM_CLASSES, _G), 0)
    oh3 = jnp.where(cidx == lab, 1.0, 0.0)
    clscol = jax.lax.dot_general(clsfull, oh3, (((1,), (0,)), ((), ())),
                                 precision=_HIGH,
                                 preferred_element_type=jnp.float32)

    m_full = m_ref[...]
    cost = _CLS_W * clscol + (_PTS_W / _P) * m_full

    minv = jnp.min(cost, axis=0, keepdims=True)
    qid = jax.lax.broadcasted_iota(jnp.int32, (_NQ, _G), 0).astype(jnp.float32)
    aq = jnp.min(jnp.where(cost == minv, qid, jnp.float32(_NQ)),
                 axis=0, keepdims=True)

    id32 = jnp.where(
        jax.lax.broadcasted_iota(jnp.int32, (_G, _G), 0)
        == jax.lax.broadcasted_iota(jnp.int32, (_G, _G), 1), 1.0, 0.0)
    aq_col = jax.lax.dot_general(id32, aq, (((1,), (1,)), ((), ())),
                                 precision=_HIGH,
                                 preferred_element_type=jnp.float32)
    later = (jax.lax.broadcasted_iota(jnp.int32, (_G, _G), 0)
             > jax.lax.broadcasted_iota(jnp.int32, (_G, _G), 1))
    eqm = jnp.where((aq_col == aq) & later, 1.0, 0.0)
    dup = jnp.max(eqm, axis=0, keepdims=True)
    win = 1.0 - dup

    oh = jnp.where(qid == aq, 1.0, 0.0)
    m_at = jnp.sum(oh * m_full, axis=0, keepdims=True)
    shift = jnp.sum(oh * ord_ref[...], axis=0, keepdims=True)
    cls_at = (minv - (_PTS_W / _P) * m_at) * (1.0 / _CLS_W)

    pos_cnt = jnp.sum(win)
    cls_num = _CLS_W * (neg_sum + jnp.sum(win * cls_at))
    pts_num = _PTS_W * jnp.sum(win * m_at)

    predpts = jax.lax.dot_general(oh, pts_ref[0], (((0,), (0,)), ((), ())),
                                  preferred_element_type=jnp.float32)
    gsrow = jax.lax.broadcasted_iota(jnp.int32, (_GS, 1), 0)
    gmod = (gsrow & (_G - 1)).astype(jnp.float32)
    gidx = jax.lax.broadcasted_iota(jnp.int32, (1, _G), 1).astype(jnp.float32)
    sdiv = jax.lax.shift_right_logical(gsrow, 5).astype(jnp.float32)
    ohs = jnp.where((gmod == gidx) & (sdiv == shift), 1.0, 0.0)
    tgt = jax.lax.dot_general(ohs, gtn_ref[...], (((0,), (1,)), ((), ())),
                              preferred_element_type=jnp.float32)

    dcol = jax.lax.broadcasted_iota(jnp.int32, (1, _D), 1)
    off_l = jnp.where(dcol % 2 == 0, _PC[0], _PC[1]).astype(jnp.float32)
    scl_l = jnp.where(dcol % 2 == 0, pw, ph).astype(jnp.float32)
    pred_den = predpts * scl_l + off_l
    tgt_den = tgt * scl_l + off_l

    pd = pred_den[:, 2:_D] - pred_den[:, 0:_D - 2]
    td = tgt_den[:, 2:_D] - tgt_den[:, 0:_D - 2]
    prod = pd * td
    pp = pd * pd
    tt = td * td
    dot2 = prod[:, 0:_D - 3] + prod[:, 1:_D - 2]
    pp2 = pp[:, 0:_D - 3] + pp[:, 1:_D - 2]
    tt2 = tt[:, 0:_D - 3] + tt[:, 1:_D - 2]
    cos = dot2 / (jnp.sqrt(pp2) * jnp.sqrt(tt2) + _EPS)
    seg_lane = jax.lax.broadcasted_iota(jnp.int32, (_G, _D - 3), 1)
    valid = (seg_lane % 2) == 0
    win_col = jax.lax.dot_general(id32, win, (((1,), (1,)), ((), ())),
                                  preferred_element_type=jnp.float32)
    dir_num = _DIR_W * jnp.sum(
        jnp.where(valid, (1.0 - cos) * win_col, 0.0))

    lane = jax.lax.broadcasted_iota(jnp.int32, (1, 128), 1)
    row = (jnp.where(lane == 0, pos_cnt, 0.0)
           + jnp.where(lane == 1, cls_num, 0.0)
           + jnp.where(lane == 2, pts_num, 0.0)
           + jnp.where(lane == 3, dir_num, 0.0))
    out_ref[0] = row


@jax.jit
def kernel(cls_scores, pts_preds, gt_shifts_pts, gt_labels):
    pts = pts_preds.reshape(_B, _NQ, _D)
    gt_t = jnp.transpose(gt_shifts_pts.reshape(_B, _G, _S, _D),
                         (0, 3, 2, 1)).reshape(_B, _D, _GS)
    lab = gt_labels.astype(jnp.int32).reshape(_B, 1, _G)

    out = pl.pallas_call(
        _body,
        grid=(_B,),
        in_specs=[
            pl.BlockSpec((1, _NQ, _NUM_CLASSES), lambda b: (b, 0, 0)),
            pl.BlockSpec((1, _NQ, _D), lambda b: (b, 0, 0)),
            pl.BlockSpec((1, _D, _GS), lambda b: (b, 0, 0)),
            pl.BlockSpec((1, 1, _G), lambda b: (b, 0, 0)),
        ],
        out_specs=pl.BlockSpec((1, 1, 128), lambda b: (b, 0, 0)),
        out_shape=jax.ShapeDtypeStruct((_B, 1, 128), jnp.float32),
        scratch_shapes=[
            pltpu.VMEM((_D, _GS), jnp.float32),
            pltpu.VMEM((_NQ, _G), jnp.float32),
            pltpu.VMEM((_NQ, _G), jnp.float32),
        ],
    )(cls_scores, pts, gt_t, lab)

    s = jnp.sum(out[:, 0, :4], axis=0)
    num_pos = jnp.maximum(s[0], 1.0)
    return (s[1] + s[2] + s[3]) / num_pos


# speedup vs baseline: 2.3433x; 1.0645x over previous
import functools
import jax
import jax.numpy as jnp
from jax.experimental import pallas as pl
from jax.experimental.pallas import tpu as pltpu

_NUM_CLASSES = 3
_PC = [-15.0, -30.0, -2.0, 15.0, 30.0, 2.0]
_CLS_W = 2.0
_PTS_W = 5.0
_DIR_W = 0.005
_ALPHA = 0.25
_GAMMA = 2.0
_EPS = 1e-12

_B, _NQ, _G, _S, _P = 4, 512, 32, 20, 20
_D = 2 * _P
_GS = _G * _S
_QT = 128
_HIGH = jax.lax.Precision.HIGHEST


def _body(cls_ref, pts_ref, gtt_ref, lab_ref, out_ref, gtn_ref, m_ref, ord_ref):
    pw = _PC[3] - _PC[0]
    ph = _PC[4] - _PC[1]

    drow = jax.lax.broadcasted_iota(jnp.int32, (_D, 1), 0)
    off_d = jnp.where(drow % 2 == 0, _PC[0], _PC[1]).astype(jnp.float32)
    scl_d = jnp.where(drow % 2 == 0, pw, ph).astype(jnp.float32)
    gtn_ref[...] = (gtt_ref[0] - off_d) / scl_d

    s_low = (jax.lax.broadcasted_iota(jnp.int32, (_QT, 128), 1) // _G)
    s_low = s_low.astype(jnp.float32)

    def one_tile(base):
        pts_t = pts_ref[0, pl.ds(base, _QT), :]
        acc = jnp.zeros((_QT, _GS), jnp.float32)
        for d in range(_D):
            acc = acc + jnp.abs(pts_t[:, d:d + 1] - gtn_ref[d:d + 1, :])
        m = acc[:, 0:128]
        sv = s_low
        for k in range(1, 5):
            sl = acc[:, 128 * k:128 * (k + 1)]
            upd = sl < m
            m = jnp.where(upd, sl, m)
            sv = jnp.where(upd, s_low + jnp.float32(4 * k), sv)
        for sh in (64, 96):
            mr = pltpu.roll(m, sh, 1)
            sr = pltpu.roll(sv, sh, 1)
            take = (mr < m) | ((mr == m) & (sr < sv))
            m = jnp.where(take, mr, m)
            sv = jnp.where(take, sr, sv)
        m_ref[pl.ds(base, _QT), :] = m[:, 0:_G]
        ord_ref[pl.ds(base, _QT), :] = sv[:, 0:_G]

    for i in range(_NQ // _QT):
        one_tile(i * _QT)

    x = cls_ref[0]
    p = jax.nn.sigmoid(x)
    negc = -jnp.log(1.0 - p + _EPS) * (p ** _GAMMA) * (1.0 - _ALPHA)
    posc = -jnp.log(p + _EPS) * ((1.0 - p) ** _GAMMA) * _ALPHA
    clsfull = posc - negc
    neg_sum = jnp.sum(negc)

    lab = lab_ref[0]
    cidx = jax.lax.broadcasted_iota(jnp.int32, (_NUM_CLASSES, _G), 0)
    oh3 = jnp.where(cidx == lab, 1.0, 0.0)
    clscol = jax.lax.dot_general(clsfull, oh3, (((1,), (0,)), ((), ())),
                                 precision=_HIGH,
                                 preferred_element_type=jnp.float32)

    m_full = m_ref[...]
    cost = _CLS_W * clscol + (_PTS_W / _P) * m_full

    minv = jnp.min(cost, axis=0, keepdims=True)
    qid = jax.lax.broadcasted_iota(jnp.int32, (_NQ, _G), 0).astype(jnp.float32)
    aq = jnp.min(jnp.where(cost == minv, qid, jnp.float32(_NQ)),
                 axis=0, keepdims=True)

    id32 = jnp.where(
        jax.lax.broadcasted_iota(jnp.int32, (_G, _G), 0)
        == jax.lax.broadcasted_iota(jnp.int32, (_G, _G), 1), 1.0, 0.0)
    aq_col = jax.lax.dot_general(id32, aq, (((1,), (1,)), ((), ())),
                                 precision=_HIGH,
                                 preferred_element_type=jnp.float32)
    later = (jax.lax.broadcasted_iota(jnp.int32, (_G, _G), 0)
             > jax.lax.broadcasted_iota(jnp.int32, (_G, _G), 1))
    eqm = jnp.where((aq_col == aq) & later, 1.0, 0.0)
    dup = jnp.max(eqm, axis=0, keepdims=True)
    win = 1.0 - dup

    oh = jnp.where(qid == aq, 1.0, 0.0)
    m_at = jnp.sum(oh * m_full, axis=0, keepdims=True)
    shift = jnp.sum(oh * ord_ref[...], axis=0, keepdims=True)
    cls_at = (minv - (_PTS_W / _P) * m_at) * (1.0 / _CLS_W)

    pos_cnt = jnp.sum(win)
    cls_num = _CLS_W * (neg_sum + jnp.sum(win * cls_at))
    pts_num = _PTS_W * jnp.sum(win * m_at)

    predpts = jax.lax.dot_general(oh, pts_ref[0], (((0,), (0,)), ((), ())),
                                  preferred_element_type=jnp.float32)
    gsrow = jax.lax.broadcasted_iota(jnp.int32, (_GS, 1), 0)
    gmod = (gsrow & (_G - 1)).astype(jnp.float32)
    gidx = jax.lax.broadcasted_iota(jnp.int32, (1, _G), 1).astype(jnp.float32)
    sdiv = jax.lax.shift_right_logical(gsrow, 5).astype(jnp.float32)
    ohs = jnp.where((gmod == gidx) & (sdiv == shift), 1.0, 0.0)
    tgt = jax.lax.dot_general(ohs, gtn_ref[...], (((0,), (1,)), ((), ())),
                              preferred_element_type=jnp.float32)

    dcol = jax.lax.broadcasted_iota(jnp.int32, (1, _D), 1)
    off_l = jnp.where(dcol % 2 == 0, _PC[0], _PC[1]).astype(jnp.float32)
    scl_l = jnp.where(dcol % 2 == 0, pw, ph).astype(jnp.float32)
    pred_den = predpts * scl_l + off_l
    tgt_den = tgt * scl_l + off_l

    pd = pred_den[:, 2:_D] - pred_den[:, 0:_D - 2]
    td = tgt_den[:, 2:_D] - tgt_den[:, 0:_D - 2]
    prod = pd * td
    pp = pd * pd
    tt = td * td
    dot2 = prod[:, 0:_D - 3] + prod[:, 1:_D - 2]
    pp2 = pp[:, 0:_D - 3] + pp[:, 1:_D - 2]
    tt2 = tt[:, 0:_D - 3] + tt[:, 1:_D - 2]
    cos = dot2 / (jnp.sqrt(pp2) * jnp.sqrt(tt2) + _EPS)
    seg_lane = jax.lax.broadcasted_iota(jnp.int32, (_G, _D - 3), 1)
    valid = (seg_lane % 2) == 0
    win_col = jax.lax.dot_general(id32, win, (((1,), (1,)), ((), ())),
                                  preferred_element_type=jnp.float32)
    dir_num = _DIR_W * jnp.sum(
        jnp.where(valid, (1.0 - cos) * win_col, 0.0))

    lane = jax.lax.broadcasted_iota(jnp.int32, (1, 128), 1)
    row = (jnp.where(lane == 0, pos_cnt, 0.0)
           + jnp.where(lane == 1, cls_num, 0.0)
           + jnp.where(lane == 2, pts_num, 0.0)
           + jnp.where(lane == 3, dir_num, 0.0))
    out_ref[0] = row


@jax.jit
def kernel(cls_scores, pts_preds, gt_shifts_pts, gt_labels):
    pts = pts_preds.reshape(_B, _NQ, _D)
    gt_t = jnp.transpose(gt_shifts_pts.reshape(_B, _G, _S, _D),
                         (0, 3, 2, 1)).reshape(_B, _D, _GS)
    lab = gt_labels.astype(jnp.int32).reshape(_B, 1, _G)

    out = pl.pallas_call(
        _body,
        grid=(_B,),
        in_specs=[
            pl.BlockSpec((1, _NQ, _NUM_CLASSES), lambda b: (b, 0, 0)),
            pl.BlockSpec((1, _NQ, _D), lambda b: (b, 0, 0)),
            pl.BlockSpec((1, _D, _GS), lambda b: (b, 0, 0)),
            pl.BlockSpec((1, 1, _G), lambda b: (b, 0, 0)),
        ],
        out_specs=pl.BlockSpec((1, 1, 128), lambda b: (b, 0, 0)),
        out_shape=jax.ShapeDtypeStruct((_B, 1, 128), jnp.float32),
        scratch_shapes=[
            pltpu.VMEM((_D, _GS), jnp.float32),
            pltpu.VMEM((_NQ, _G), jnp.float32),
            pltpu.VMEM((_NQ, _G), jnp.float32),
        ],
    )(cls_scores, pts, gt_t, lab)

    s = jnp.sum(out[:, 0, :4], axis=0)
    num_pos = jnp.maximum(s[0], 1.0)
    return (s[1] + s[2] + s[3]) / num_pos


# speedup vs baseline: 2.4773x; 1.0572x over previous
import functools
import jax
import jax.numpy as jnp
from jax.experimental import pallas as pl
from jax.experimental.pallas import tpu as pltpu

_NUM_CLASSES = 3
_PC = [-15.0, -30.0, -2.0, 15.0, 30.0, 2.0]
_CLS_W = 2.0
_PTS_W = 5.0
_DIR_W = 0.005
_ALPHA = 0.25
_GAMMA = 2.0
_EPS = 1e-12

_B, _NQ, _G, _S, _P = 4, 512, 32, 20, 20
_D = 2 * _P
_GS = _G * _S
_QT = 256
_HIGH = jax.lax.Precision.HIGHEST


def _body(cls_ref, pts_ref, gtt_ref, lab_ref, out_ref, gtn_ref, m_ref, ord_ref):
    pw = _PC[3] - _PC[0]
    ph = _PC[4] - _PC[1]

    drow = jax.lax.broadcasted_iota(jnp.int32, (_D, 1), 0)
    off_d = jnp.where(drow % 2 == 0, _PC[0], _PC[1]).astype(jnp.float32)
    scl_d = jnp.where(drow % 2 == 0, pw, ph).astype(jnp.float32)
    gtn_ref[...] = (gtt_ref[0] - off_d) / scl_d

    s_low = (jax.lax.broadcasted_iota(jnp.int32, (_QT, 128), 1) // _G)
    s_low = s_low.astype(jnp.float32)

    def one_tile(base):
        pts_t = pts_ref[0, pl.ds(base, _QT), :]
        acc = jnp.zeros((_QT, _GS), jnp.float32)
        for d in range(_D):
            acc = acc + jnp.abs(pts_t[:, d:d + 1] - gtn_ref[d:d + 1, :])
        m = acc[:, 0:128]
        sv = s_low
        for k in range(1, 5):
            sl = acc[:, 128 * k:128 * (k + 1)]
            upd = sl < m
            m = jnp.where(upd, sl, m)
            sv = jnp.where(upd, s_low + jnp.float32(4 * k), sv)
        for sh in (64, 96):
            mr = pltpu.roll(m, sh, 1)
            sr = pltpu.roll(sv, sh, 1)
            take = (mr < m) | ((mr == m) & (sr < sv))
            m = jnp.where(take, mr, m)
            sv = jnp.where(take, sr, sv)
        m_ref[pl.ds(base, _QT), :] = m[:, 0:_G]
        ord_ref[pl.ds(base, _QT), :] = sv[:, 0:_G]

    for i in range(_NQ // _QT):
        one_tile(i * _QT)

    x = cls_ref[0]
    p = jax.nn.sigmoid(x)
    negc = -jnp.log(1.0 - p + _EPS) * (p ** _GAMMA) * (1.0 - _ALPHA)
    posc = -jnp.log(p + _EPS) * ((1.0 - p) ** _GAMMA) * _ALPHA
    clsfull = posc - negc
    neg_sum = jnp.sum(negc)

    lab = lab_ref[0]
    cidx = jax.lax.broadcasted_iota(jnp.int32, (_NUM_CLASSES, _G), 0)
    oh3 = jnp.where(cidx == lab, 1.0, 0.0)
    clscol = jax.lax.dot_general(clsfull, oh3, (((1,), (0,)), ((), ())),
                                 precision=_HIGH,
                                 preferred_element_type=jnp.float32)

    m_full = m_ref[...]
    cost = _CLS_W * clscol + (_PTS_W / _P) * m_full

    minv = jnp.min(cost, axis=0, keepdims=True)
    qid = jax.lax.broadcasted_iota(jnp.int32, (_NQ, _G), 0).astype(jnp.float32)
    aq = jnp.min(jnp.where(cost == minv, qid, jnp.float32(_NQ)),
                 axis=0, keepdims=True)

    id32 = jnp.where(
        jax.lax.broadcasted_iota(jnp.int32, (_G, _G), 0)
        == jax.lax.broadcasted_iota(jnp.int32, (_G, _G), 1), 1.0, 0.0)
    aq_col = jax.lax.dot_general(id32, aq, (((1,), (1,)), ((), ())),
                                 precision=_HIGH,
                                 preferred_element_type=jnp.float32)
    later = (jax.lax.broadcasted_iota(jnp.int32, (_G, _G), 0)
             > jax.lax.broadcasted_iota(jnp.int32, (_G, _G), 1))
    eqm = jnp.where((aq_col == aq) & later, 1.0, 0.0)
    dup = jnp.max(eqm, axis=0, keepdims=True)
    win = 1.0 - dup

    oh = jnp.where(qid == aq, 1.0, 0.0)
    m_at = jnp.sum(oh * m_full, axis=0, keepdims=True)
    shift = jnp.sum(oh * ord_ref[...], axis=0, keepdims=True)
    cls_at = (minv - (_PTS_W / _P) * m_at) * (1.0 / _CLS_W)

    pos_cnt = jnp.sum(win)
    cls_num = _CLS_W * (neg_sum + jnp.sum(win * cls_at))
    pts_num = _PTS_W * jnp.sum(win * m_at)

    predpts = jax.lax.dot_general(oh, pts_ref[0], (((0,), (0,)), ((), ())),
                                  preferred_element_type=jnp.float32)
    gsrow = jax.lax.broadcasted_iota(jnp.int32, (_GS, 1), 0)
    gmod = (gsrow & (_G - 1)).astype(jnp.float32)
    gidx = jax.lax.broadcasted_iota(jnp.int32, (1, _G), 1).astype(jnp.float32)
    sdiv = jax.lax.shift_right_logical(gsrow, 5).astype(jnp.float32)
    ohs = jnp.where((gmod == gidx) & (sdiv == shift), 1.0, 0.0)
    tgt = jax.lax.dot_general(ohs, gtn_ref[...], (((0,), (1,)), ((), ())),
                              preferred_element_type=jnp.float32)

    dcol = jax.lax.broadcasted_iota(jnp.int32, (1, _D), 1)
    off_l = jnp.where(dcol % 2 == 0, _PC[0], _PC[1]).astype(jnp.float32)
    scl_l = jnp.where(dcol % 2 == 0, pw, ph).astype(jnp.float32)
    pred_den = predpts * scl_l + off_l
    tgt_den = tgt * scl_l + off_l

    pd = pred_den[:, 2:_D] - pred_den[:, 0:_D - 2]
    td = tgt_den[:, 2:_D] - tgt_den[:, 0:_D - 2]
    prod = pd * td
    pp = pd * pd
    tt = td * td
    dot2 = prod[:, 0:_D - 3] + prod[:, 1:_D - 2]
    pp2 = pp[:, 0:_D - 3] + pp[:, 1:_D - 2]
    tt2 = tt[:, 0:_D - 3] + tt[:, 1:_D - 2]
    cos = dot2 / (jnp.sqrt(pp2) * jnp.sqrt(tt2) + _EPS)
    seg_lane = jax.lax.broadcasted_iota(jnp.int32, (_G, _D - 3), 1)
    valid = (seg_lane % 2) == 0
    win_col = jax.lax.dot_general(id32, win, (((1,), (1,)), ((), ())),
                                  preferred_element_type=jnp.float32)
    dir_num = _DIR_W * jnp.sum(
        jnp.where(valid, (1.0 - cos) * win_col, 0.0))

    lane = jax.lax.broadcasted_iota(jnp.int32, (1, 128), 1)
    row = (jnp.where(lane == 0, pos_cnt, 0.0)
           + jnp.where(lane == 1, cls_num, 0.0)
           + jnp.where(lane == 2, pts_num, 0.0)
           + jnp.where(lane == 3, dir_num, 0.0))
    out_ref[0] = row


@jax.jit
def kernel(cls_scores, pts_preds, gt_shifts_pts, gt_labels):
    pts = pts_preds.reshape(_B, _NQ, _D)
    gt_t = jnp.transpose(gt_shifts_pts.reshape(_B, _G, _S, _D),
                         (0, 3, 2, 1)).reshape(_B, _D, _GS)
    lab = gt_labels.astype(jnp.int32).reshape(_B, 1, _G)

    out = pl.pallas_call(
        _body,
        grid=(_B,),
        in_specs=[
            pl.BlockSpec((1, _NQ, _NUM_CLASSES), lambda b: (b, 0, 0)),
            pl.BlockSpec((1, _NQ, _D), lambda b: (b, 0, 0)),
            pl.BlockSpec((1, _D, _GS), lambda b: (b, 0, 0)),
            pl.BlockSpec((1, 1, _G), lambda b: (b, 0, 0)),
        ],
        out_specs=pl.BlockSpec((1, 1, 128), lambda b: (b, 0, 0)),
        out_shape=jax.ShapeDtypeStruct((_B, 1, 128), jnp.float32),
        scratch_shapes=[
            pltpu.VMEM((_D, _GS), jnp.float32),
            pltpu.VMEM((_NQ, _G), jnp.float32),
            pltpu.VMEM((_NQ, _G), jnp.float32),
        ],
    )(cls_scores, pts, gt_t, lab)

    s = jnp.sum(out[:, 0, :4], axis=0)
    num_pos = jnp.maximum(s[0], 1.0)
    return (s[1] + s[2] + s[3]) / num_pos


# speedup vs baseline: 2.6059x; 1.0519x over previous
import functools
import jax
import jax.numpy as jnp
from jax.experimental import pallas as pl
from jax.experimental.pallas import tpu as pltpu

_NUM_CLASSES = 3
_PC = [-15.0, -30.0, -2.0, 15.0, 30.0, 2.0]
_CLS_W = 2.0
_PTS_W = 5.0
_DIR_W = 0.005
_ALPHA = 0.25
_GAMMA = 2.0
_EPS = 1e-12

_B, _NQ, _G, _S, _P = 4, 512, 32, 20, 20
_D = 2 * _P
_GS = _G * _S
_QT = 512
_HIGH = jax.lax.Precision.HIGHEST


def _body(cls_ref, pts_ref, gtt_ref, lab_ref, out_ref, gtn_ref, m_ref, ord_ref):
    pw = _PC[3] - _PC[0]
    ph = _PC[4] - _PC[1]

    drow = jax.lax.broadcasted_iota(jnp.int32, (_D, 1), 0)
    off_d = jnp.where(drow % 2 == 0, _PC[0], _PC[1]).astype(jnp.float32)
    scl_d = jnp.where(drow % 2 == 0, pw, ph).astype(jnp.float32)
    gtn_ref[...] = (gtt_ref[0] - off_d) / scl_d

    s_low = (jax.lax.broadcasted_iota(jnp.int32, (_QT, 128), 1) // _G)
    s_low = s_low.astype(jnp.float32)

    def one_tile(base):
        pts_t = pts_ref[0, pl.ds(base, _QT), :]
        acc = jnp.zeros((_QT, _GS), jnp.float32)
        for d in range(_D):
            acc = acc + jnp.abs(pts_t[:, d:d + 1] - gtn_ref[d:d + 1, :])
        m = acc[:, 0:128]
        sv = s_low
        for k in range(1, 5):
            sl = acc[:, 128 * k:128 * (k + 1)]
            upd = sl < m
            m = jnp.where(upd, sl, m)
            sv = jnp.where(upd, s_low + jnp.float32(4 * k), sv)
        for sh in (64, 96):
            mr = pltpu.roll(m, sh, 1)
            sr = pltpu.roll(sv, sh, 1)
            take = (mr < m) | ((mr == m) & (sr < sv))
            m = jnp.where(take, mr, m)
            sv = jnp.where(take, sr, sv)
        m_ref[pl.ds(base, _QT), :] = m[:, 0:_G]
        ord_ref[pl.ds(base, _QT), :] = sv[:, 0:_G]

    for i in range(_NQ // _QT):
        one_tile(i * _QT)

    x = cls_ref[0]
    p = jax.nn.sigmoid(x)
    negc = -jnp.log(1.0 - p + _EPS) * (p ** _GAMMA) * (1.0 - _ALPHA)
    posc = -jnp.log(p + _EPS) * ((1.0 - p) ** _GAMMA) * _ALPHA
    clsfull = posc - negc
    neg_sum = jnp.sum(negc)

    lab = lab_ref[0]
    cidx = jax.lax.broadcasted_iota(jnp.int32, (_NUM_CLASSES, _G), 0)
    oh3 = jnp.where(cidx == lab, 1.0, 0.0)
    clscol = jax.lax.dot_general(clsfull, oh3, (((1,), (0,)), ((), ())),
                                 precision=_HIGH,
                                 preferred_element_type=jnp.float32)

    m_full = m_ref[...]
    cost = _CLS_W * clscol + (_PTS_W / _P) * m_full

    minv = jnp.min(cost, axis=0, keepdims=True)
    qid = jax.lax.broadcasted_iota(jnp.int32, (_NQ, _G), 0).astype(jnp.float32)
    aq = jnp.min(jnp.where(cost == minv, qid, jnp.float32(_NQ)),
                 axis=0, keepdims=True)

    id32 = jnp.where(
        jax.lax.broadcasted_iota(jnp.int32, (_G, _G), 0)
        == jax.lax.broadcasted_iota(jnp.int32, (_G, _G), 1), 1.0, 0.0)
    aq_col = jax.lax.dot_general(id32, aq, (((1,), (1,)), ((), ())),
                                 precision=_HIGH,
                                 preferred_element_type=jnp.float32)
    later = (jax.lax.broadcasted_iota(jnp.int32, (_G, _G), 0)
             > jax.lax.broadcasted_iota(jnp.int32, (_G, _G), 1))
    eqm = jnp.where((aq_col == aq) & later, 1.0, 0.0)
    dup = jnp.max(eqm, axis=0, keepdims=True)
    win = 1.0 - dup

    oh = jnp.where(qid == aq, 1.0, 0.0)
    m_at = jnp.sum(oh * m_full, axis=0, keepdims=True)
    shift = jnp.sum(oh * ord_ref[...], axis=0, keepdims=True)
    cls_at = (minv - (_PTS_W / _P) * m_at) * (1.0 / _CLS_W)

    pos_cnt = jnp.sum(win)
    cls_num = _CLS_W * (neg_sum + jnp.sum(win * cls_at))
    pts_num = _PTS_W * jnp.sum(win * m_at)

    predpts = jax.lax.dot_general(oh, pts_ref[0], (((0,), (0,)), ((), ())),
                                  preferred_element_type=jnp.float32)
    gsrow = jax.lax.broadcasted_iota(jnp.int32, (_GS, 1), 0)
    gmod = (gsrow & (_G - 1)).astype(jnp.float32)
    gidx = jax.lax.broadcasted_iota(jnp.int32, (1, _G), 1).astype(jnp.float32)
    sdiv = jax.lax.shift_right_logical(gsrow, 5).astype(jnp.float32)
    ohs = jnp.where((gmod == gidx) & (sdiv == shift), 1.0, 0.0)
    tgt = jax.lax.dot_general(ohs, gtn_ref[...], (((0,), (1,)), ((), ())),
                              preferred_element_type=jnp.float32)

    dcol = jax.lax.broadcasted_iota(jnp.int32, (1, _D), 1)
    off_l = jnp.where(dcol % 2 == 0, _PC[0], _PC[1]).astype(jnp.float32)
    scl_l = jnp.where(dcol % 2 == 0, pw, ph).astype(jnp.float32)
    pred_den = predpts * scl_l + off_l
    tgt_den = tgt * scl_l + off_l

    pd = pred_den[:, 2:_D] - pred_den[:, 0:_D - 2]
    td = tgt_den[:, 2:_D] - tgt_den[:, 0:_D - 2]
    prod = pd * td
    pp = pd * pd
    tt = td * td
    dot2 = prod[:, 0:_D - 3] + prod[:, 1:_D - 2]
    pp2 = pp[:, 0:_D - 3] + pp[:, 1:_D - 2]
    tt2 = tt[:, 0:_D - 3] + tt[:, 1:_D - 2]
    cos = dot2 / (jnp.sqrt(pp2) * jnp.sqrt(tt2) + _EPS)
    seg_lane = jax.lax.broadcasted_iota(jnp.int32, (_G, _D - 3), 1)
    valid = (seg_lane % 2) == 0
    win_col = jax.lax.dot_general(id32, win, (((1,), (1,)), ((), ())),
                                  preferred_element_type=jnp.float32)
    dir_num = _DIR_W * jnp.sum(
        jnp.where(valid, (1.0 - cos) * win_col, 0.0))

    lane = jax.lax.broadcasted_iota(jnp.int32, (1, 128), 1)
    row = (jnp.where(lane == 0, pos_cnt, 0.0)
           + jnp.where(lane == 1, cls_num, 0.0)
           + jnp.where(lane == 2, pts_num, 0.0)
           + jnp.where(lane == 3, dir_num, 0.0))
    out_ref[0] = row


@jax.jit
def kernel(cls_scores, pts_preds, gt_shifts_pts, gt_labels):
    pts = pts_preds.reshape(_B, _NQ, _D)
    gt_t = jnp.transpose(gt_shifts_pts.reshape(_B, _G, _S, _D),
                         (0, 3, 2, 1)).reshape(_B, _D, _GS)
    lab = gt_labels.astype(jnp.int32).reshape(_B, 1, _G)

    out = pl.pallas_call(
        _body,
        grid=(_B,),
        in_specs=[
            pl.BlockSpec((1, _NQ, _NUM_CLASSES), lambda b: (b, 0, 0)),
            pl.BlockSpec((1, _NQ, _D), lambda b: (b, 0, 0)),
            pl.BlockSpec((1, _D, _GS), lambda b: (b, 0, 0)),
            pl.BlockSpec((1, 1, _G), lambda b: (b, 0, 0)),
        ],
        out_specs=pl.BlockSpec((1, 1, 128), lambda b: (b, 0, 0)),
        out_shape=jax.ShapeDtypeStruct((_B, 1, 128), jnp.float32),
        scratch_shapes=[
            pltpu.VMEM((_D, _GS), jnp.float32),
            pltpu.VMEM((_NQ, _G), jnp.float32),
            pltpu.VMEM((_NQ, _G), jnp.float32),
        ],
    )(cls_scores, pts, gt_t, lab)

    s = jnp.sum(out[:, 0, :4], axis=0)
    num_pos = jnp.maximum(s[0], 1.0)
    return (s[1] + s[2] + s[3]) / num_pos
